# Initial kernel scaffold; baseline (speedup 1.0000x reference)
import functools
import jax
import jax.numpy as jnp
from jax import lax
from jax.experimental import pallas as pl
from jax.experimental.pallas import tpu as pltpu
from jax.experimental.pallas import tpu_sc as plsc

NC = 2
NS = 16
NW = NC * NS
LANES = 16
GROUP = 128
GPB = 8
BATCH = GROUP * GPB


def _make_sc_kernel(n_nodes: int, d: int, p_per_tile: int):
    assert d == 32
    n_batches = p_per_tile // BATCH
    e_total = NW * p_per_tile

    mesh = plsc.VectorSubcoreMesh(
        core_axis_name="c", subcore_axis_name="s",
        num_cores=NC, num_subcores=NS)

    @functools.partial(
        pl.kernel,
        out_type=jax.ShapeDtypeStruct((e_total,), jnp.float32),
        mesh=mesh,
        scratch_types=[
            pltpu.VMEM((GPB, GROUP), jnp.int32),
            pltpu.VMEM((GPB, GROUP), jnp.int32),
            pltpu.VMEM((BATCH, 32), jnp.float32),
            pltpu.VMEM((BATCH, 32), jnp.float32),
            pltpu.VMEM((BATCH,), jnp.float32),
            pltpu.SemaphoreType.DMA,
        ],
    )
    def k(z_hbm, src_hbm, dst_hbm, out_hbm, sidx, didx, srow, drow, obuf, sem):
        wid = lax.axis_index("s") * NC + lax.axis_index("c")
        idx_row0 = wid * (p_per_tile // GROUP)
        out0 = wid * p_per_tile
        lane = lax.iota(jnp.int32, LANES)

        def batch_body(b, carry):
            pltpu.sync_copy(src_hbm.at[pl.ds(idx_row0 + b * GPB, GPB), :], sidx)
            pltpu.sync_copy(dst_hbm.at[pl.ds(idx_row0 + b * GPB, GPB), :], didx)
            copies = []
            for j in range(GPB):
                copies.append(pltpu.async_copy(
                    z_hbm.at[sidx.at[j]], srow.at[pl.ds(j * GROUP, GROUP), :], sem))
                copies.append(pltpu.async_copy(
                    z_hbm.at[didx.at[j]], drow.at[pl.ds(j * GROUP, GROUP), :], sem))
            for c in copies:
                c.wait()

            def group_body(g, c2):
                acc = jnp.zeros((LANES,), jnp.float32)
                for e in range(LANES):
                    q = g * LANES + e
                    s = (srow[q, pl.ds(0, 16)] * drow[q, pl.ds(0, 16)]
                         + srow[q, pl.ds(16, 16)] * drow[q, pl.ds(16, 16)])
                    acc = jnp.where(lane == e, jnp.sum(s), acc)
                obuf[pl.ds(g * LANES, LANES)] = 1.0 / (1.0 + jnp.exp(-acc))
                return c2

            lax.fori_loop(0, BATCH // LANES, group_body, 0, unroll=False)
            pltpu.sync_copy(obuf, out_hbm.at[pl.ds(out0 + b * BATCH, BATCH)])
            return carry

        lax.fori_loop(0, n_batches, batch_body, 0, unroll=False)

    return k


def kernel(z, edge_index):
    n_nodes, d = z.shape
    e = edge_index.shape[1]
    p_per_tile = -(-e // (NW * BATCH)) * BATCH
    e_total = NW * p_per_tile

    src = edge_index[0].astype(jnp.int32)
    dst = edge_index[1].astype(jnp.int32)
    pad = e_total - e
    if pad:
        zeros = jnp.zeros((pad,), jnp.int32)
        src = jnp.concatenate([src, zeros])
        dst = jnp.concatenate([dst, zeros])
    src = src.reshape(e_total // GROUP, GROUP)
    dst = dst.reshape(e_total // GROUP, GROUP)

    out = _make_sc_kernel(n_nodes, d, p_per_tile)(z.astype(jnp.float32), src, dst)
    return out[:e]


# speedup vs baseline: 16.5975x; 16.5975x over previous
import functools
import jax
import jax.numpy as jnp
from jax import lax
from jax.experimental import pallas as pl
from jax.experimental.pallas import tpu as pltpu
from jax.experimental.pallas import tpu_sc as plsc

NC = 2
NS = 16
NW = NC * NS
LANES = 16
GROUP = 128
GPB = 4
BATCH = GROUP * GPB


def _make_sc_kernel(n_nodes: int, d: int, p_per_tile: int):
    assert d == 32
    n_batches = p_per_tile // BATCH
    assert n_batches >= 4 and n_batches % 2 == 0
    e_total = NW * p_per_tile

    mesh = plsc.VectorSubcoreMesh(
        core_axis_name="c", subcore_axis_name="s",
        num_cores=NC, num_subcores=NS)

    @functools.partial(
        pl.kernel,
        out_type=jax.ShapeDtypeStruct((e_total,), jnp.float32),
        mesh=mesh,
        compiler_params=pltpu.CompilerParams(
            needs_layout_passes=False, use_tc_tiling_on_sc=False),
        scratch_types=[
            pltpu.VMEM((2, GPB, GROUP), jnp.int32),
            pltpu.VMEM((2, GPB, GROUP), jnp.int32),
            pltpu.VMEM((2, BATCH, 32), jnp.float32),
            pltpu.VMEM((2, BATCH, 32), jnp.float32),
            pltpu.VMEM((2, BATCH), jnp.float32),
            pltpu.SemaphoreType.DMA,
            pltpu.SemaphoreType.DMA,
            pltpu.SemaphoreType.DMA,
            pltpu.SemaphoreType.DMA,
            pltpu.SemaphoreType.DMA,
            pltpu.SemaphoreType.DMA,
        ],
    )
    def k(z_hbm, src_hbm, dst_hbm, out_hbm, sidx, didx, srow, drow, obuf,
          gsem0, gsem1, isem0, isem1, osem0, osem1):
        wid = lax.axis_index("s") * NC + lax.axis_index("c")
        idx_row0 = wid * (p_per_tile // GROUP)
        out0 = wid * p_per_tile
        lane = lax.iota(jnp.int32, LANES)
        gsems = (gsem0, gsem1)
        isems = (isem0, isem1)
        osems = (osem0, osem1)

        def fire_idx(b, p):
            pltpu.async_copy(src_hbm.at[pl.ds(idx_row0 + b * GPB, GPB), :],
                             sidx.at[p], isems[p])
            pltpu.async_copy(dst_hbm.at[pl.ds(idx_row0 + b * GPB, GPB), :],
                             didx.at[p], isems[p])

        def drain_idx(p):
            pltpu.make_async_copy(src_hbm.at[pl.ds(idx_row0, GPB), :],
                                  sidx.at[p], isems[p]).wait()
            pltpu.make_async_copy(dst_hbm.at[pl.ds(idx_row0, GPB), :],
                                  didx.at[p], isems[p]).wait()

        def fire_gathers(p):
            for j in range(GPB):
                pltpu.async_copy(z_hbm.at[sidx.at[p, j]],
                                 srow.at[p, pl.ds(j * GROUP, GROUP), :],
                                 gsems[p])
                pltpu.async_copy(z_hbm.at[didx.at[p, j]],
                                 drow.at[p, pl.ds(j * GROUP, GROUP), :],
                                 gsems[p])

        def drain_gathers(p):
            for j in range(GPB):
                pltpu.make_async_copy(z_hbm.at[sidx.at[p, j]],
                                      srow.at[p, pl.ds(j * GROUP, GROUP), :],
                                      gsems[p]).wait()
                pltpu.make_async_copy(z_hbm.at[didx.at[p, j]],
                                      drow.at[p, pl.ds(j * GROUP, GROUP), :],
                                      gsems[p]).wait()

        def compute(b, p):
            @pl.when(b >= 2)
            def _():
                pltpu.make_async_copy(obuf.at[p],
                                      out_hbm.at[pl.ds(out0, BATCH)],
                                      osems[p]).wait()

            def group_body(g, c2):
                acc = jnp.zeros((LANES,), jnp.float32)
                for e in range(LANES):
                    q = g * LANES + e
                    s = (srow[p, q, pl.ds(0, 16)] * drow[p, q, pl.ds(0, 16)]
                         + srow[p, q, pl.ds(16, 16)] * drow[p, q, pl.ds(16, 16)])
                    acc = jnp.where(lane == e, jnp.sum(s), acc)
                obuf[p, pl.ds(g * LANES, LANES)] = 1.0 / (1.0 + jnp.exp(-acc))
                return c2

            lax.fori_loop(0, BATCH // LANES, group_body, 0, unroll=False)
            pltpu.async_copy(obuf.at[p],
                             out_hbm.at[pl.ds(out0 + b * BATCH, BATCH)],
                             osems[p])

        fire_idx(0, 0)
        fire_idx(1, 1)
        drain_idx(0)
        fire_gathers(0)

        def step(b, p):
            drain_idx(1 - p)
            fire_gathers(1 - p)
            drain_gathers(p)
            fire_idx(b + 2, p)
            compute(b, p)

        def loop_body(i, carry):
            b0 = i * 2
            step(b0, 0)
            step(b0 + 1, 1)
            return carry

        lax.fori_loop(0, (n_batches - 2) // 2, loop_body, 0, unroll=False)
        drain_idx(1)
        fire_gathers(1)
        drain_gathers(0)
        compute(n_batches - 2, 0)
        drain_gathers(1)
        compute(n_batches - 1, 1)
        pltpu.make_async_copy(obuf.at[0], out_hbm.at[pl.ds(out0, BATCH)],
                              osems[0]).wait()
        pltpu.make_async_copy(obuf.at[1], out_hbm.at[pl.ds(out0, BATCH)],
                              osems[1]).wait()

    return k


def kernel(z, edge_index):
    n_nodes, d = z.shape
    e = edge_index.shape[1]
    p_per_tile = -(-e // (NW * BATCH)) * BATCH
    e_total = NW * p_per_tile

    src = edge_index[0].astype(jnp.int32)
    dst = edge_index[1].astype(jnp.int32)
    pad = e_total - e
    if pad:
        zeros = jnp.zeros((pad,), jnp.int32)
        src = jnp.concatenate([src, zeros])
        dst = jnp.concatenate([dst, zeros])
    src = src.reshape(e_total // GROUP, GROUP)
    dst = dst.reshape(e_total // GROUP, GROUP)

    out = _make_sc_kernel(n_nodes, d, p_per_tile)(z.astype(jnp.float32), src, dst)
    return out[:e]


# speedup vs baseline: 20.1529x; 1.2142x over previous
import functools
import jax
import jax.numpy as jnp
from jax import lax
from jax.experimental import pallas as pl
from jax.experimental.pallas import tpu as pltpu
from jax.experimental.pallas import tpu_sc as plsc

NC = 2
NS = 16
NW = NC * NS
LANES = 16
GROUP = 128
GPB = 4
BATCH = GROUP * GPB


def _make_sc_kernel(n_nodes: int, d: int, p_per_tile: int):
    assert d == 32
    n_batches = p_per_tile // BATCH
    assert n_batches >= 4 and n_batches % 2 == 0
    e_total = NW * p_per_tile

    mesh = plsc.VectorSubcoreMesh(
        core_axis_name="c", subcore_axis_name="s",
        num_cores=NC, num_subcores=NS)

    @functools.partial(
        pl.kernel,
        out_type=jax.ShapeDtypeStruct((e_total,), jnp.float32),
        mesh=mesh,
        compiler_params=pltpu.CompilerParams(
            needs_layout_passes=False, use_tc_tiling_on_sc=False),
        scratch_types=[
            pltpu.VMEM((2, GPB, GROUP), jnp.int32),
            pltpu.VMEM((2, GPB, GROUP), jnp.int32),
            pltpu.VMEM((2, BATCH, 32), jnp.bfloat16),
            pltpu.VMEM((2, BATCH, 32), jnp.bfloat16),
            pltpu.VMEM((2, BATCH), jnp.float32),
            pltpu.SemaphoreType.DMA,
            pltpu.SemaphoreType.DMA,
            pltpu.SemaphoreType.DMA,
            pltpu.SemaphoreType.DMA,
            pltpu.SemaphoreType.DMA,
            pltpu.SemaphoreType.DMA,
        ],
    )
    def k(z_hbm, src_hbm, dst_hbm, out_hbm, sidx, didx, srow, drow, obuf,
          gsem0, gsem1, isem0, isem1, osem0, osem1):
        wid = lax.axis_index("s") * NC + lax.axis_index("c")
        idx_row0 = wid * (p_per_tile // GROUP)
        out0 = wid * p_per_tile
        lane = lax.iota(jnp.int32, LANES)
        gsems = (gsem0, gsem1)
        isems = (isem0, isem1)
        osems = (osem0, osem1)

        def fire_idx(b, p):
            pltpu.async_copy(src_hbm.at[pl.ds(idx_row0 + b * GPB, GPB), :],
                             sidx.at[p], isems[p])
            pltpu.async_copy(dst_hbm.at[pl.ds(idx_row0 + b * GPB, GPB), :],
                             didx.at[p], isems[p])

        def drain_idx(p):
            pltpu.make_async_copy(src_hbm.at[pl.ds(idx_row0, GPB), :],
                                  sidx.at[p], isems[p]).wait()
            pltpu.make_async_copy(dst_hbm.at[pl.ds(idx_row0, GPB), :],
                                  didx.at[p], isems[p]).wait()

        def fire_gathers(p):
            for j in range(GPB):
                pltpu.async_copy(z_hbm.at[sidx.at[p, j]],
                                 srow.at[p, pl.ds(j * GROUP, GROUP), :],
                                 gsems[p])
                pltpu.async_copy(z_hbm.at[didx.at[p, j]],
                                 drow.at[p, pl.ds(j * GROUP, GROUP), :],
                                 gsems[p])

        def drain_gathers(p):
            for j in range(GPB):
                pltpu.make_async_copy(z_hbm.at[sidx.at[p, j]],
                                      srow.at[p, pl.ds(j * GROUP, GROUP), :],
                                      gsems[p]).wait()
                pltpu.make_async_copy(z_hbm.at[didx.at[p, j]],
                                      drow.at[p, pl.ds(j * GROUP, GROUP), :],
                                      gsems[p]).wait()

        def compute(b, p):
            @pl.when(b >= 2)
            def _():
                pltpu.make_async_copy(obuf.at[p],
                                      out_hbm.at[pl.ds(out0, BATCH)],
                                      osems[p]).wait()

            def group_body(g, c2):
                acc = jnp.zeros((LANES,), jnp.float32)
                for e in range(LANES):
                    q = g * LANES + e
                    prod = srow[p, q, :] * drow[p, q, :]
                    pa, pb = plsc.unpack(prod, format=plsc.PackFormat.INTERLEAVED,
                                         preferred_element_type=jnp.float32)
                    acc = jnp.where(lane == e, jnp.sum(pa + pb), acc)
                obuf[p, pl.ds(g * LANES, LANES)] = 1.0 / (1.0 + jnp.exp(-acc))
                return c2

            lax.fori_loop(0, BATCH // LANES, group_body, 0, unroll=False)
            pltpu.async_copy(obuf.at[p],
                             out_hbm.at[pl.ds(out0 + b * BATCH, BATCH)],
                             osems[p])

        fire_idx(0, 0)
        fire_idx(1, 1)
        drain_idx(0)
        fire_gathers(0)

        def step(b, p):
            drain_idx(1 - p)
            fire_gathers(1 - p)
            drain_gathers(p)
            fire_idx(b + 2, p)
            compute(b, p)

        def loop_body(i, carry):
            b0 = i * 2
            step(b0, 0)
            step(b0 + 1, 1)
            return carry

        lax.fori_loop(0, (n_batches - 2) // 2, loop_body, 0, unroll=False)
        drain_idx(1)
        fire_gathers(1)
        drain_gathers(0)
        compute(n_batches - 2, 0)
        drain_gathers(1)
        compute(n_batches - 1, 1)
        pltpu.make_async_copy(obuf.at[0], out_hbm.at[pl.ds(out0, BATCH)],
                              osems[0]).wait()
        pltpu.make_async_copy(obuf.at[1], out_hbm.at[pl.ds(out0, BATCH)],
                              osems[1]).wait()

    return k


def kernel(z, edge_index):
    n_nodes, d = z.shape
    e = edge_index.shape[1]
    p_per_tile = -(-e // (NW * BATCH)) * BATCH
    e_total = NW * p_per_tile

    src = edge_index[0].astype(jnp.int32)
    dst = edge_index[1].astype(jnp.int32)
    pad = e_total - e
    if pad:
        zeros = jnp.zeros((pad,), jnp.int32)
        src = jnp.concatenate([src, zeros])
        dst = jnp.concatenate([dst, zeros])
    src = src.reshape(e_total // GROUP, GROUP)
    dst = dst.reshape(e_total // GROUP, GROUP)

    out = _make_sc_kernel(n_nodes, d, p_per_tile)(
        z.astype(jnp.bfloat16), src, dst)
    return out[:e]


# speedup vs baseline: 24.0917x; 1.1954x over previous
import functools
import jax
import jax.numpy as jnp
from jax import lax
from jax.experimental import pallas as pl
from jax.experimental.pallas import tpu as pltpu
from jax.experimental.pallas import tpu_sc as plsc

NC = 2
NS = 16
NW = NC * NS
LANES = 16
GROUP = 128
GPB = 4
BATCH = GROUP * GPB


def _make_sc_kernel(n_nodes: int, d: int, e_edges: int):
    assert d == 32
    assert e_edges % NW == 0
    per_tile = e_edges // NW
    n_full = per_tile // BATCH
    tail = per_tile - n_full * BATCH
    assert n_full >= 4 and n_full % 2 == 1
    assert tail % LANES == 0
    tail_groups = [GROUP] * (tail // GROUP)
    if tail % GROUP:
        tail_groups.append(tail % GROUP)

    mesh = plsc.VectorSubcoreMesh(
        core_axis_name="c", subcore_axis_name="s",
        num_cores=NC, num_subcores=NS)

    @functools.partial(
        pl.kernel,
        out_type=jax.ShapeDtypeStruct((e_edges,), jnp.float32),
        mesh=mesh,
        compiler_params=pltpu.CompilerParams(
            needs_layout_passes=False, use_tc_tiling_on_sc=False),
        scratch_types=[
            pltpu.VMEM((2 * GPB, GROUP), jnp.int32),
            pltpu.VMEM((2 * GPB, GROUP), jnp.int32),
            pltpu.VMEM((2, BATCH, 32), jnp.bfloat16),
            pltpu.VMEM((2, BATCH, 32), jnp.bfloat16),
            pltpu.VMEM((2, BATCH), jnp.float32),
            pltpu.SemaphoreType.DMA,
            pltpu.SemaphoreType.DMA,
            pltpu.SemaphoreType.DMA,
            pltpu.SemaphoreType.DMA,
            pltpu.SemaphoreType.DMA,
            pltpu.SemaphoreType.DMA,
        ],
    )
    def k(z_hbm, src_hbm, dst_hbm, out_hbm, sidx, didx, srow, drow, obuf,
          gsem0, gsem1, isem0, isem1, osem0, osem1):
        wid = lax.axis_index("s") * NC + lax.axis_index("c")
        base = wid * per_tile
        lane = lax.iota(jnp.int32, LANES)
        gsems = (gsem0, gsem1)
        isems = (isem0, isem1)
        osems = (osem0, osem1)

        def fire_idx(b, p):
            for j in range(GPB):
                off = base + b * BATCH + j * GROUP
                pltpu.async_copy(src_hbm.at[pl.ds(off, GROUP)],
                                 sidx.at[p * GPB + j], isems[p])
                pltpu.async_copy(dst_hbm.at[pl.ds(off, GROUP)],
                                 didx.at[p * GPB + j], isems[p])

        def drain_idx(p):
            for j in range(GPB):
                pltpu.make_async_copy(src_hbm.at[pl.ds(base, GROUP)],
                                      sidx.at[p * GPB + j], isems[p]).wait()
                pltpu.make_async_copy(dst_hbm.at[pl.ds(base, GROUP)],
                                      didx.at[p * GPB + j], isems[p]).wait()

        def fire_gathers(p):
            for j in range(GPB):
                pltpu.async_copy(z_hbm.at[sidx.at[p * GPB + j]],
                                 srow.at[p, pl.ds(j * GROUP, GROUP), :],
                                 gsems[p])
                pltpu.async_copy(z_hbm.at[didx.at[p * GPB + j]],
                                 drow.at[p, pl.ds(j * GROUP, GROUP), :],
                                 gsems[p])

        def drain_gathers(p):
            for j in range(GPB):
                pltpu.make_async_copy(z_hbm.at[sidx.at[p * GPB + j]],
                                      srow.at[p, pl.ds(j * GROUP, GROUP), :],
                                      gsems[p]).wait()
                pltpu.make_async_copy(z_hbm.at[didx.at[p * GPB + j]],
                                      drow.at[p, pl.ds(j * GROUP, GROUP), :],
                                      gsems[p]).wait()

        def dot_groups(p, n_groups):
            def group_body(g, c2):
                acc = jnp.zeros((LANES,), jnp.float32)
                for e in range(LANES):
                    q = g * LANES + e
                    prod = srow[p, q, :] * drow[p, q, :]
                    pa, pb = plsc.unpack(
                        prod, format=plsc.PackFormat.INTERLEAVED,
                        preferred_element_type=jnp.float32)
                    acc = jnp.where(lane == e, jnp.sum(pa + pb), acc)
                obuf[p, pl.ds(g * LANES, LANES)] = 1.0 / (1.0 + jnp.exp(-acc))
                return c2
            lax.fori_loop(0, n_groups, group_body, 0, unroll=False)

        def compute(b, p):
            @pl.when(b >= 2)
            def _():
                pltpu.make_async_copy(obuf.at[p],
                                      out_hbm.at[pl.ds(base, BATCH)],
                                      osems[p]).wait()
            dot_groups(p, BATCH // LANES)
            pltpu.async_copy(obuf.at[p],
                             out_hbm.at[pl.ds(base + b * BATCH, BATCH)],
                             osems[p])

        if tail:
            t0 = base + n_full * BATCH
            for j, gsz in enumerate(tail_groups):
                off = t0 + j * GROUP
                pltpu.async_copy(src_hbm.at[pl.ds(off, gsz)],
                                 sidx.at[j, pl.ds(0, gsz)], isems[0])
                pltpu.async_copy(dst_hbm.at[pl.ds(off, gsz)],
                                 didx.at[j, pl.ds(0, gsz)], isems[0])
            for j, gsz in enumerate(tail_groups):
                pltpu.make_async_copy(src_hbm.at[pl.ds(base, gsz)],
                                      sidx.at[j, pl.ds(0, gsz)],
                                      isems[0]).wait()
                pltpu.make_async_copy(dst_hbm.at[pl.ds(base, gsz)],
                                      didx.at[j, pl.ds(0, gsz)],
                                      isems[0]).wait()
            for j, gsz in enumerate(tail_groups):
                pltpu.async_copy(z_hbm.at[sidx.at[j, pl.ds(0, gsz)]],
                                 srow.at[0, pl.ds(j * GROUP, gsz), :],
                                 gsems[0])
                pltpu.async_copy(z_hbm.at[didx.at[j, pl.ds(0, gsz)]],
                                 drow.at[0, pl.ds(j * GROUP, gsz), :],
                                 gsems[0])
            for j, gsz in enumerate(tail_groups):
                pltpu.make_async_copy(z_hbm.at[sidx.at[j, pl.ds(0, gsz)]],
                                      srow.at[0, pl.ds(j * GROUP, gsz), :],
                                      gsems[0]).wait()
                pltpu.make_async_copy(z_hbm.at[didx.at[j, pl.ds(0, gsz)]],
                                      drow.at[0, pl.ds(j * GROUP, gsz), :],
                                      gsems[0]).wait()
            dot_groups(0, tail // LANES)
            pltpu.async_copy(obuf.at[0, pl.ds(0, tail)],
                             out_hbm.at[pl.ds(t0, tail)], osems[0])
            pltpu.make_async_copy(obuf.at[0, pl.ds(0, tail)],
                                  out_hbm.at[pl.ds(t0, tail)],
                                  osems[0]).wait()

        fire_idx(0, 0)
        fire_idx(1, 1)
        drain_idx(0)
        fire_gathers(0)

        def step(b, p, prefetch_idx=True):
            drain_idx(1 - p)
            fire_gathers(1 - p)
            drain_gathers(p)
            if prefetch_idx:
                fire_idx(b + 2, p)
            compute(b, p)

        def loop_body(i, carry):
            b0 = i * 2
            step(b0, 0)
            step(b0 + 1, 1)
            return carry

        lax.fori_loop(0, (n_full - 3) // 2, loop_body, 0, unroll=False)
        step(n_full - 3, 0)
        step(n_full - 2, 1, prefetch_idx=False)
        drain_gathers(0)
        compute(n_full - 1, 0)
        pltpu.make_async_copy(obuf.at[0], out_hbm.at[pl.ds(base, BATCH)],
                              osems[0]).wait()
        pltpu.make_async_copy(obuf.at[1], out_hbm.at[pl.ds(base, BATCH)],
                              osems[1]).wait()

    return k


def kernel(z, edge_index):
    n_nodes, d = z.shape
    e = edge_index.shape[1]
    src = edge_index[0].astype(jnp.int32)
    dst = edge_index[1].astype(jnp.int32)
    return _make_sc_kernel(n_nodes, d, e)(z.astype(jnp.bfloat16), src, dst)


# speedup vs baseline: 26.9500x; 1.1186x over previous
import functools
import jax
import jax.numpy as jnp
from jax import lax
from jax.experimental import pallas as pl
from jax.experimental.pallas import tpu as pltpu
from jax.experimental.pallas import tpu_sc as plsc

NC = 2
NS = 16
NW = NC * NS
LANES = 16
GROUP = 128
GPB = 4
BATCH = GROUP * GPB
DEPTH = 3


def _make_sc_kernel(n_nodes: int, d: int, e_edges: int):
    assert d == 32
    assert e_edges % NW == 0
    per_tile = e_edges // NW
    n_full = per_tile // BATCH
    tail = per_tile - n_full * BATCH
    assert n_full >= 6
    assert tail % LANES == 0
    tail_groups = [GROUP] * (tail // GROUP)
    if tail % GROUP:
        tail_groups.append(tail % GROUP)

    mesh = plsc.VectorSubcoreMesh(
        core_axis_name="c", subcore_axis_name="s",
        num_cores=NC, num_subcores=NS)

    @functools.partial(
        pl.kernel,
        out_type=jax.ShapeDtypeStruct((e_edges,), jnp.float32),
        mesh=mesh,
        compiler_params=pltpu.CompilerParams(
            needs_layout_passes=False, use_tc_tiling_on_sc=False),
        scratch_types=[
            pltpu.VMEM((DEPTH * GPB, GROUP), jnp.int32),
            pltpu.VMEM((DEPTH * GPB, GROUP), jnp.int32),
            pltpu.VMEM((DEPTH, BATCH, 32), jnp.bfloat16),
            pltpu.VMEM((DEPTH, BATCH, 32), jnp.bfloat16),
            pltpu.VMEM((DEPTH, BATCH), jnp.float32),
            [pltpu.SemaphoreType.DMA] * DEPTH,
            [pltpu.SemaphoreType.DMA] * DEPTH,
            [pltpu.SemaphoreType.DMA] * DEPTH,
        ],
    )
    def k(z_hbm, ei_hbm, out_hbm, sidx, didx, srow, drow, obuf,
          gsems, isems, osems):
        wid = lax.axis_index("s") * NC + lax.axis_index("c")
        base = wid * per_tile
        lane = lax.iota(jnp.int32, LANES)

        def fire_idx(b, r):
            for j in range(GPB):
                off = base + b * BATCH + j * GROUP
                pltpu.async_copy(ei_hbm.at[0, pl.ds(off, GROUP)],
                                 sidx.at[r * GPB + j], isems[r])
                pltpu.async_copy(ei_hbm.at[1, pl.ds(off, GROUP)],
                                 didx.at[r * GPB + j], isems[r])

        def drain_idx(r):
            for j in range(GPB):
                pltpu.make_async_copy(ei_hbm.at[0, pl.ds(base, GROUP)],
                                      sidx.at[r * GPB + j], isems[r]).wait()
                pltpu.make_async_copy(ei_hbm.at[1, pl.ds(base, GROUP)],
                                      didx.at[r * GPB + j], isems[r]).wait()

        def fire_gathers(r):
            for j in range(GPB):
                pltpu.async_copy(z_hbm.at[sidx.at[r * GPB + j]],
                                 srow.at[r, pl.ds(j * GROUP, GROUP), :],
                                 gsems[r])
                pltpu.async_copy(z_hbm.at[didx.at[r * GPB + j]],
                                 drow.at[r, pl.ds(j * GROUP, GROUP), :],
                                 gsems[r])

        def drain_gathers(r):
            for j in range(GPB):
                pltpu.make_async_copy(z_hbm.at[sidx.at[r * GPB + j]],
                                      srow.at[r, pl.ds(j * GROUP, GROUP), :],
                                      gsems[r]).wait()
                pltpu.make_async_copy(z_hbm.at[didx.at[r * GPB + j]],
                                      drow.at[r, pl.ds(j * GROUP, GROUP), :],
                                      gsems[r]).wait()

        def dot_groups(r, n_groups):
            def group_body(g, c2):
                acc = jnp.zeros((LANES,), jnp.float32)
                for e in range(LANES):
                    q = g * LANES + e
                    prod = srow[r, q, :] * drow[r, q, :]
                    pa, pb = plsc.unpack(
                        prod, format=plsc.PackFormat.INTERLEAVED,
                        preferred_element_type=jnp.float32)
                    acc = jnp.where(lane == e, jnp.sum(pa + pb), acc)
                obuf[r, pl.ds(g * LANES, LANES)] = 1.0 / (1.0 + jnp.exp(-acc))
                return c2
            lax.fori_loop(0, n_groups, group_body, 0, unroll=False)

        def compute(b, r):
            @pl.when(b >= DEPTH)
            def _():
                pltpu.make_async_copy(obuf.at[r],
                                      out_hbm.at[pl.ds(base, BATCH)],
                                      osems[r]).wait()
            dot_groups(r, BATCH // LANES)
            pltpu.async_copy(obuf.at[r],
                             out_hbm.at[pl.ds(base + b * BATCH, BATCH)],
                             osems[r])

        if tail:
            t0 = base + n_full * BATCH
            for j, gsz in enumerate(tail_groups):
                off = t0 + j * GROUP
                pltpu.async_copy(ei_hbm.at[0, pl.ds(off, gsz)],
                                 sidx.at[j, pl.ds(0, gsz)], isems[0])
                pltpu.async_copy(ei_hbm.at[1, pl.ds(off, gsz)],
                                 didx.at[j, pl.ds(0, gsz)], isems[0])
            for j, gsz in enumerate(tail_groups):
                pltpu.make_async_copy(ei_hbm.at[0, pl.ds(base, gsz)],
                                      sidx.at[j, pl.ds(0, gsz)],
                                      isems[0]).wait()
                pltpu.make_async_copy(ei_hbm.at[1, pl.ds(base, gsz)],
                                      didx.at[j, pl.ds(0, gsz)],
                                      isems[0]).wait()
            for j, gsz in enumerate(tail_groups):
                pltpu.async_copy(z_hbm.at[sidx.at[j, pl.ds(0, gsz)]],
                                 srow.at[0, pl.ds(j * GROUP, gsz), :],
                                 gsems[0])
                pltpu.async_copy(z_hbm.at[didx.at[j, pl.ds(0, gsz)]],
                                 drow.at[0, pl.ds(j * GROUP, gsz), :],
                                 gsems[0])
            for j, gsz in enumerate(tail_groups):
                pltpu.make_async_copy(z_hbm.at[sidx.at[j, pl.ds(0, gsz)]],
                                      srow.at[0, pl.ds(j * GROUP, gsz), :],
                                      gsems[0]).wait()
                pltpu.make_async_copy(z_hbm.at[didx.at[j, pl.ds(0, gsz)]],
                                      drow.at[0, pl.ds(j * GROUP, gsz), :],
                                      gsems[0]).wait()
            dot_groups(0, tail // LANES)
            pltpu.async_copy(obuf.at[0, pl.ds(0, tail)],
                             out_hbm.at[pl.ds(t0, tail)], osems[0])
            pltpu.make_async_copy(obuf.at[0, pl.ds(0, tail)],
                                  out_hbm.at[pl.ds(t0, tail)],
                                  osems[0]).wait()

        fire_idx(0, 0)
        fire_idx(1, 1)
        fire_idx(2, 2)
        drain_idx(0)
        fire_gathers(0)
        drain_idx(1)
        fire_gathers(1)

        def stepper(b, r, do_g, do_i):
            if do_g:
                drain_idx((r + 2) % DEPTH)
                fire_gathers((r + 2) % DEPTH)
            drain_gathers(r)
            if do_i:
                fire_idx(b + 3, r)
            compute(b, r)

        m3 = ((n_full - 3) // 3) * 3

        def loop_body(i, carry):
            b0 = i * 3
            stepper(b0, 0, True, True)
            stepper(b0 + 1, 1, True, True)
            stepper(b0 + 2, 2, True, True)
            return carry

        lax.fori_loop(0, m3 // 3, loop_body, 0, unroll=False)
        for b in range(m3, n_full):
            stepper(b, b % 3, b + 2 <= n_full - 1, b + 3 <= n_full - 1)
        for r in range(DEPTH):
            pltpu.make_async_copy(obuf.at[r], out_hbm.at[pl.ds(base, BATCH)],
                                  osems[r]).wait()

    return k


def kernel(z, edge_index):
    n_nodes, d = z.shape
    e = edge_index.shape[1]
    ei = edge_index.astype(jnp.int32)
    return _make_sc_kernel(n_nodes, d, e)(z.astype(jnp.bfloat16), ei)
